# baseline (speedup 1.0000x reference)
import jax
import jax.numpy as jnp
from jax.experimental import pallas as pl
from jax.experimental.pallas import tpu as pltpu

E = 8
H = 128
BLK = 2048


def _mlp_block(x_ref, W1_ref, b1_ref, W2_ref, b2_ref, W3_ref, b3_ref, o_ref):
    x = x_ref[...]
    t = x + 1.0
    c = (t >= 1.0).astype(jnp.int32)
    ev = c[:, 2:3] + 2 * c[:, 1:2] + 4 * c[:, 0:1]
    x0 = x[:, 0:1]
    x1 = x[:, 1:2]
    x2 = x[:, 2:3]
    y = jnp.zeros((x.shape[0], 1), jnp.float32)
    for e in range(E):
        h1 = jax.nn.relu(
            x0 * W1_ref[e, 0:1, :] + x1 * W1_ref[e, 1:2, :] + x2 * W1_ref[e, 2:3, :]
            + b1_ref[e:e + 1, :]
        )
        h2 = jax.nn.relu(
            jnp.dot(h1, W2_ref[e], preferred_element_type=jnp.float32)
            + b2_ref[e:e + 1, :]
        )
        ye = jnp.sum(h2 * W3_ref[e], axis=1, keepdims=True) + b3_ref[e:e + 1, :]
        y = jnp.where(ev == e, ye, y)
    o_ref[...] = y


def kernel(x, W1, b1, W2, b2, W3, b3):
    N = x.shape[0]
    W3r = W3.reshape(E, 1, H)
    grid = (N // BLK,)
    out = pl.pallas_call(
        _mlp_block,
        grid=grid,
        in_specs=[
            pl.BlockSpec((BLK, 3), lambda g: (g, 0)),
            pl.BlockSpec((E, 3, H), lambda g: (0, 0, 0)),
            pl.BlockSpec((E, H), lambda g: (0, 0)),
            pl.BlockSpec((E, H, H), lambda g: (0, 0, 0)),
            pl.BlockSpec((E, H), lambda g: (0, 0)),
            pl.BlockSpec((E, 1, H), lambda g: (0, 0, 0)),
            pl.BlockSpec((E, 1), lambda g: (0, 0)),
        ],
        out_specs=pl.BlockSpec((BLK, 1), lambda g: (g, 0)),
        out_shape=jax.ShapeDtypeStruct((N, 1), jnp.float32),
    )(x, W1, b1, W2, b2, W3r, b3)
    return out
